# Initial kernel scaffold; baseline (speedup 1.0000x reference)
import jax
import jax.numpy as jnp
from jax.experimental import pallas as pl


def kernel(x):
    raise NotImplementedError("write your pallas kernel here")


# speedup vs baseline: 2.6510x; 2.6510x over previous
import jax
import jax.numpy as jnp
from jax.experimental import pallas as pl
from jax.experimental.pallas import tpu as pltpu

_R = 512
_C = 256


def _body(x_ref, tri_ref, o_ref, carry_ref):
    j = pl.program_id(1)

    @pl.when(j == 0)
    def _():
        carry_ref[...] = jnp.zeros_like(carry_ref)

    y = jax.lax.dot(x_ref[...], tri_ref[...],
                    preferred_element_type=jnp.float32)
    y = y + carry_ref[...]
    o_ref[...] = y
    carry_ref[...] = y[:, _C - 1:_C]


def kernel(x):
    M, N = x.shape
    tri = jnp.triu(jnp.ones((_C, _C), jnp.float32))
    return pl.pallas_call(
        _body,
        grid=(M // _R, N // _C),
        in_specs=[
            pl.BlockSpec((_R, _C), lambda i, j: (i, j)),
            pl.BlockSpec((_C, _C), lambda i, j: (0, 0)),
        ],
        out_specs=pl.BlockSpec((_R, _C), lambda i, j: (i, j)),
        out_shape=jax.ShapeDtypeStruct((M, N), x.dtype),
        scratch_shapes=[pltpu.VMEM((_R, 1), jnp.float32)],
        compiler_params=pltpu.CompilerParams(
            dimension_semantics=("parallel", "arbitrary"),
        ),
    )(x, tri)
